# Initial kernel scaffold; baseline (speedup 1.0000x reference)
import functools

import jax
import jax.numpy as jnp
from jax import lax
from jax.experimental import pallas as pl
from jax.experimental.pallas import tpu as pltpu
from jax.experimental.pallas import tpu_sc as plsc

N = 10000
E = 320000
F_IN = 128
H = 128
C = 40
CP = 48

NC = 2
NS = 16
NW = NC * NS
K = 128

ROWS_PER_TILE = N // NS


def _make_spmm(feat: int, e_pad: int):
    epw = e_pad // NW
    chunks = epw // K
    assert chunks * K * NW == e_pad

    mesh = plsc.VectorSubcoreMesh(core_axis_name="c", subcore_axis_name="s")

    @functools.partial(
        pl.kernel,
        out_type=jax.ShapeDtypeStruct((NC * N, feat), jnp.float32),
        mesh=mesh,
        scratch_types=[
            pltpu.VMEM((K, feat), jnp.float32),
            pltpu.VMEM((K,), jnp.int32),
            pltpu.VMEM((K,), jnp.int32),
            pltpu.VMEM((K,), jnp.float32),
            pltpu.VMEM_SHARED((N, feat), jnp.float32),
            pltpu.SemaphoreType.DMA,
        ],
    )
    def spmm(sup_hbm, src_hbm, dst_hbm, w_hbm, out_hbm,
             rows_v, src_v, dst_v, w_v, acc_sh, sem):
        cid = lax.axis_index("c")
        sid = lax.axis_index("s")
        wid = sid * NC + cid

        def zero_row(i, _):
            for f in range(feat // 16):
                rows_v[i, pl.ds(f * 16, 16)] = jnp.zeros((16,), jnp.float32)
            return 0
        lax.fori_loop(0, K, zero_row, 0)

        base_row = sid * ROWS_PER_TILE
        nfull = ROWS_PER_TILE // K
        rem = ROWS_PER_TILE - nfull * K
        for j in range(nfull):
            pltpu.sync_copy(rows_v, acc_sh.at[pl.ds(base_row + j * K, K)])
        if rem:
            pltpu.sync_copy(rows_v.at[pl.ds(0, rem)],
                            acc_sh.at[pl.ds(base_row + nfull * K, rem)])
        plsc.subcore_barrier()

        ebase0 = wid * epw

        def chunk(g, _):
            eb = ebase0 + g * K
            pltpu.sync_copy(src_hbm.at[pl.ds(eb, K)], src_v)
            pltpu.sync_copy(dst_hbm.at[pl.ds(eb, K)], dst_v)
            pltpu.sync_copy(w_hbm.at[pl.ds(eb, K)], w_v)
            pltpu.async_copy(sup_hbm.at[src_v], rows_v, sem).wait()

            def scale(e, _):
                wv = jnp.full((16,), w_v[e], jnp.float32)
                for f in range(feat // 16):
                    sl = pl.ds(f * 16, 16)
                    rows_v[e, sl] = rows_v[e, sl] * wv
                return 0
            lax.fori_loop(0, K, scale, 0)

            pltpu.sync_copy(rows_v, acc_sh.at[dst_v], add=True)
            return 0
        lax.fori_loop(0, chunks, chunk, 0)
        plsc.subcore_barrier()

        out_base = cid * N + base_row
        pltpu.sync_copy(acc_sh.at[pl.ds(base_row, ROWS_PER_TILE)],
                        out_hbm.at[pl.ds(out_base, ROWS_PER_TILE)])

    return spmm


_E_PAD = ((E + NW * K - 1) // (NW * K)) * NW * K
_spmm_h = _make_spmm(H, _E_PAD)
_spmm_c = _make_spmm(CP, _E_PAD)

_BLK = 400
_GRID = N // _BLK


def _mm1_body(x_ref, w_ref, o_ref):
    o_ref[...] = jnp.dot(x_ref[...], w_ref[...],
                         preferred_element_type=jnp.float32)


def _layer2_body(p0_ref, p1_ref, b1_ref, w2_ref, o_ref):
    h = jnp.maximum(p0_ref[...] + p1_ref[...] + b1_ref[...], 0.0)
    o_ref[...] = jnp.dot(h, w2_ref[...], preferred_element_type=jnp.float32)


def _out_body(p0_ref, p1_ref, b2_ref, o_ref):
    o = p0_ref[...] + p1_ref[...] + b2_ref[...]
    col = lax.broadcasted_iota(jnp.int32, o.shape, 1)
    o = jnp.where(col < C, o, -jnp.inf)
    m = jnp.max(o, axis=1, keepdims=True)
    ls = jnp.log(jnp.sum(jnp.exp(o - m), axis=1, keepdims=True))
    res = o - m - ls
    o_ref[...] = res[:, :C]


def kernel(x, edge_index, edge_weight, W1, b1, W2, b2):
    dst = edge_index[0]
    src = edge_index[1]
    pad = _E_PAD - E
    src_p = jnp.pad(src, (0, pad))
    dst_p = jnp.pad(dst, (0, pad))
    w_p = jnp.pad(edge_weight, (0, pad))
    W2p = jnp.pad(W2, ((0, 0), (0, CP - C)))
    b2p = jnp.pad(b2, (0, CP - C))

    support1 = pl.pallas_call(
        _mm1_body,
        grid=(_GRID,),
        in_specs=[
            pl.BlockSpec((_BLK, F_IN), lambda i: (i, 0)),
            pl.BlockSpec((F_IN, H), lambda i: (0, 0)),
        ],
        out_specs=pl.BlockSpec((_BLK, H), lambda i: (i, 0)),
        out_shape=jax.ShapeDtypeStruct((N, H), jnp.float32),
    )(x, W1)

    parts1 = _spmm_h(support1, src_p, dst_p, w_p)

    support2 = pl.pallas_call(
        _layer2_body,
        grid=(_GRID,),
        in_specs=[
            pl.BlockSpec((_BLK, H), lambda i: (i, 0)),
            pl.BlockSpec((_BLK, H), lambda i: (i + _GRID, 0)),
            pl.BlockSpec((1, H), lambda i: (0, 0)),
            pl.BlockSpec((H, CP), lambda i: (0, 0)),
        ],
        out_specs=pl.BlockSpec((_BLK, CP), lambda i: (i, 0)),
        out_shape=jax.ShapeDtypeStruct((N, CP), jnp.float32),
    )(parts1, parts1, b1.reshape(1, H), W2p)

    parts2 = _spmm_c(support2, src_p, dst_p, w_p)

    out = pl.pallas_call(
        _out_body,
        grid=(_GRID,),
        in_specs=[
            pl.BlockSpec((_BLK, CP), lambda i: (i, 0)),
            pl.BlockSpec((_BLK, CP), lambda i: (i + _GRID, 0)),
            pl.BlockSpec((1, CP), lambda i: (0, 0)),
        ],
        out_specs=pl.BlockSpec((_BLK, C), lambda i: (i, 0)),
        out_shape=jax.ShapeDtypeStruct((N, C), jnp.float32),
    )(parts2, parts2, b2p.reshape(1, CP))

    return out


# speedup vs baseline: 4.3342x; 4.3342x over previous
import functools

import jax
import jax.numpy as jnp
from jax import lax
from jax.experimental import pallas as pl
from jax.experimental.pallas import tpu as pltpu
from jax.experimental.pallas import tpu_sc as plsc

N = 10000
E = 320000
F_IN = 128
H = 128
C = 40
CP = 48

NC = 2
NS = 16
NW = NC * NS
K = 128

N_PAD = 10112
ROWS_PER_TILE = N_PAD // NS


def _make_spmm(feat: int, e_pad: int, tc_tiling: bool = True):
    epw = e_pad // NW
    chunks = epw // K
    assert chunks * K * NW == e_pad

    mesh = plsc.VectorSubcoreMesh(core_axis_name="c", subcore_axis_name="s")

    @functools.partial(
        pl.kernel,
        out_type=jax.ShapeDtypeStruct((NC * N_PAD, feat), jnp.float32),
        mesh=mesh,
        scratch_types=[
            pltpu.VMEM((K, feat), jnp.float32),
            pltpu.VMEM((K,), jnp.int32),
            pltpu.VMEM((K,), jnp.int32),
            pltpu.VMEM((K,), jnp.float32),
            pltpu.VMEM_SHARED((N_PAD, feat), jnp.float32),
            pltpu.SemaphoreType.DMA,
        ],
        compiler_params=pltpu.CompilerParams(use_tc_tiling_on_sc=tc_tiling),
    )
    def spmm(sup_hbm, src_hbm, dst_hbm, w_hbm, out_hbm,
             rows_v, src_v, dst_v, w_v, acc_sh, sem):
        cid = lax.axis_index("c")
        sid = lax.axis_index("s")
        wid = sid * NC + cid

        def zero_row(i, _):
            for f in range(feat // 16):
                rows_v[i, pl.ds(f * 16, 16)] = jnp.zeros((16,), jnp.float32)
            return 0
        lax.fori_loop(0, K, zero_row, 0)

        base_row = sid * ROWS_PER_TILE
        nfull = ROWS_PER_TILE // K
        rem = ROWS_PER_TILE - nfull * K
        for j in range(nfull):
            pltpu.sync_copy(rows_v, acc_sh.at[pl.ds(base_row + j * K, K)])
        if rem:
            pltpu.sync_copy(rows_v.at[pl.ds(0, rem)],
                            acc_sh.at[pl.ds(base_row + nfull * K, rem)])
        plsc.subcore_barrier()

        ebase0 = wid * epw

        def chunk(g, _):
            eb = ebase0 + g * K
            pltpu.sync_copy(src_hbm.at[pl.ds(eb, K)], src_v)
            pltpu.sync_copy(dst_hbm.at[pl.ds(eb, K)], dst_v)
            pltpu.sync_copy(w_hbm.at[pl.ds(eb, K)], w_v)
            pltpu.async_copy(sup_hbm.at[src_v], rows_v, sem).wait()

            def scale16(b, _):
                w16 = w_v[pl.ds(b * 16, 16)]
                eb16 = b * 16
                for i in range(16):
                    wv = jnp.full((16,), w16[i], jnp.float32)
                    for f in range(feat // 16):
                        sl = pl.ds(f * 16, 16)
                        rows_v[eb16 + i, sl] = rows_v[eb16 + i, sl] * wv
                return 0
            lax.fori_loop(0, K // 16, scale16, 0)

            pltpu.sync_copy(rows_v, acc_sh.at[dst_v], add=True)
            return 0
        lax.fori_loop(0, chunks, chunk, 0)
        plsc.subcore_barrier()

        out_base = cid * N_PAD + base_row
        pltpu.sync_copy(acc_sh.at[pl.ds(base_row, ROWS_PER_TILE)],
                        out_hbm.at[pl.ds(out_base, ROWS_PER_TILE)])

    return spmm


_E_PAD = ((E + NW * K - 1) // (NW * K)) * NW * K
_spmm_h = _make_spmm(H, _E_PAD)
_spmm_c = _make_spmm(CP, _E_PAD, tc_tiling=False)

_BLK = 400
_GRID = N // _BLK


def _mm1_body(x_ref, w_ref, o_ref):
    o_ref[...] = jnp.dot(x_ref[...], w_ref[...],
                         preferred_element_type=jnp.float32)


def _layer2_body(p0_ref, p1_ref, b1_ref, w2_ref, o_ref):
    h = jnp.maximum(p0_ref[0] + p1_ref[0] + b1_ref[...], 0.0)
    o_ref[...] = jnp.dot(h, w2_ref[...], preferred_element_type=jnp.float32)


def _out_body(p0_ref, p1_ref, b2_ref, o_ref):
    o = p0_ref[0] + p1_ref[0] + b2_ref[...]
    col = lax.broadcasted_iota(jnp.int32, o.shape, 1)
    o = jnp.where(col < C, o, -jnp.inf)
    m = jnp.max(o, axis=1, keepdims=True)
    ls = jnp.log(jnp.sum(jnp.exp(o - m), axis=1, keepdims=True))
    res = o - m - ls
    o_ref[...] = res[:, :C]


def kernel(x, edge_index, edge_weight, W1, b1, W2, b2):
    dst = edge_index[0]
    src = edge_index[1]
    pad = _E_PAD - E
    src_p = jnp.pad(src, (0, pad))
    dst_p = jnp.pad(dst, (0, pad))
    w_p = jnp.pad(edge_weight, (0, pad))
    W2p = jnp.pad(W2, ((0, 0), (0, CP - C)))
    b2p = jnp.pad(b2, (0, CP - C))

    support1 = pl.pallas_call(
        _mm1_body,
        grid=(_GRID,),
        in_specs=[
            pl.BlockSpec((_BLK, F_IN), lambda i: (i, 0)),
            pl.BlockSpec((F_IN, H), lambda i: (0, 0)),
        ],
        out_specs=pl.BlockSpec((_BLK, H), lambda i: (i, 0)),
        out_shape=jax.ShapeDtypeStruct((N, H), jnp.float32),
    )(x, W1)

    parts1 = _spmm_h(support1, src_p, dst_p, w_p).reshape(NC, N_PAD, H)

    support2 = pl.pallas_call(
        _layer2_body,
        grid=(_GRID,),
        in_specs=[
            pl.BlockSpec((1, _BLK, H), lambda i: (0, i, 0)),
            pl.BlockSpec((1, _BLK, H), lambda i: (1, i, 0)),
            pl.BlockSpec((1, H), lambda i: (0, 0)),
            pl.BlockSpec((H, CP), lambda i: (0, 0)),
        ],
        out_specs=pl.BlockSpec((_BLK, CP), lambda i: (i, 0)),
        out_shape=jax.ShapeDtypeStruct((N, CP), jnp.float32),
    )(parts1, parts1, b1.reshape(1, H), W2p)

    parts2 = _spmm_c(support2, src_p, dst_p, w_p).reshape(NC, N_PAD, CP)

    out = pl.pallas_call(
        _out_body,
        grid=(_GRID,),
        in_specs=[
            pl.BlockSpec((1, _BLK, CP), lambda i: (0, i, 0)),
            pl.BlockSpec((1, _BLK, CP), lambda i: (1, i, 0)),
            pl.BlockSpec((1, CP), lambda i: (0, 0)),
        ],
        out_specs=pl.BlockSpec((_BLK, C), lambda i: (i, 0)),
        out_shape=jax.ShapeDtypeStruct((N, C), jnp.float32),
    )(parts2, parts2, b2p.reshape(1, CP))

    return out


# speedup vs baseline: 5.7686x; 1.3310x over previous
import functools

import jax
import jax.numpy as jnp
from jax import lax
from jax.experimental import pallas as pl
from jax.experimental.pallas import tpu as pltpu
from jax.experimental.pallas import tpu_sc as plsc

N = 10000
E = 320000
F_IN = 128
H = 128
HH = H // 2
C = 40
CP = 48

NC = 2
NS = 16
NW = NC * NS
K = 128
PH = 80

N_PAD = 10112
ROWS_PER_TILE = N_PAD // NS

_E_PAD = NS * 2 * PH * K


def _make_spmm(feat: int, feature_split: bool):
    phases = 2 if feature_split else 1

    mesh = plsc.VectorSubcoreMesh(core_axis_name="c", subcore_axis_name="s")
    n_sup = 2 if feature_split else 1

    @functools.partial(
        pl.kernel,
        out_type=jax.ShapeDtypeStruct((NC * N_PAD, feat), jnp.float32),
        mesh=mesh,
        scratch_types=[
            pltpu.VMEM((K, feat), jnp.float32),
            pltpu.VMEM((K, feat), jnp.float32),
            pltpu.VMEM((K, feat), jnp.float32),
            pltpu.VMEM((K, feat), jnp.float32),
            pltpu.VMEM((PH, 3, K), jnp.int32),
            pltpu.VMEM_SHARED((N_PAD, feat), jnp.float32),
            pltpu.SemaphoreType.DMA,
            pltpu.SemaphoreType.DMA,
            pltpu.SemaphoreType.DMA,
            pltpu.SemaphoreType.DMA,
            pltpu.SemaphoreType.DMA,
        ],
        compiler_params=pltpu.CompilerParams(use_tc_tiling_on_sc=False,
                                             needs_layout_passes=False),
    )
    def spmm(*args):
        sups = args[:n_sup]
        idx_hbm = args[n_sup]
        out_hbm = args[n_sup + 1]
        (rows0, rows1, msgs0, msgs1, idxb, acc_sh,
         gs0, gs1, ss0, ss1, isem) = args[n_sup + 2:]
        rows = (rows0, rows1)
        msgs = (msgs0, msgs1)
        gsem = (gs0, gs1)
        ssem = (ss0, ss1)
        cid = lax.axis_index("c")
        sid = lax.axis_index("s")

        if feature_split:
            cb = sid * (phases * PH)
        else:
            cb = (sid * NC + cid) * PH

        cp_i = pltpu.async_copy(idx_hbm.at[pl.ds(cb, PH)], idxb, isem)

        def zero_row(i, _):
            for f in range(feat // 16):
                msgs0[i, pl.ds(f * 16, 16)] = jnp.zeros((16,), jnp.float32)
            return 0
        lax.fori_loop(0, K, zero_row, 0)

        base_row = sid * ROWS_PER_TILE
        nfull = ROWS_PER_TILE // K
        rem = ROWS_PER_TILE - nfull * K
        for j in range(nfull):
            pltpu.sync_copy(msgs0, acc_sh.at[pl.ds(base_row + j * K, K)])
        if rem:
            pltpu.sync_copy(msgs0.at[pl.ds(0, rem)],
                            acc_sh.at[pl.ds(base_row + nfull * K, rem)])
        cp_i.wait()
        plsc.subcore_barrier()

        def scale_to(g, rv, mv):
            def scale16(b16, _):
                w16 = plsc.bitcast(idxb[g, 2, pl.ds(b16 * 16, 16)],
                                   jnp.float32)
                e0 = b16 * 16
                for i in range(16):
                    wv = jnp.full((16,), w16[i], jnp.float32)
                    for f in range(feat // 16):
                        sl = pl.ds(f * 16, 16)
                        mv[e0 + i, sl] = rv[e0 + i, sl] * wv
                return 0
            lax.fori_loop(0, K // 16, scale16, 0)

        def start_gather(g, b):
            if feature_split:
                @pl.when(cid == 0)
                def _():
                    pltpu.async_copy(sups[0].at[idxb.at[g, 0]], rows[b],
                                     gsem[b])

                @pl.when(cid == 1)
                def _():
                    pltpu.async_copy(sups[1].at[idxb.at[g, 0]], rows[b],
                                     gsem[b])
            else:
                pltpu.async_copy(sups[0].at[idxb.at[g, 0]], rows[b], gsem[b])

        def wait_gather(g, b):
            pltpu.make_async_copy(sups[0].at[idxb.at[g, 0]], rows[b],
                                  gsem[b]).wait()

        def start_scatter(g, m):
            pltpu.async_copy(msgs[m], acc_sh.at[idxb.at[g, 1]], ssem[m],
                             add=True)

        def wait_scatter(g, m):
            pltpu.make_async_copy(msgs[m], acc_sh.at[idxb.at[g, 1]],
                                  ssem[m]).wait()

        for p in range(phases):
            if p > 0:
                pltpu.sync_copy(idx_hbm.at[pl.ds(cb + p * PH, PH)], idxb)

            for b in range(2):
                start_gather(b, b)

            for u in range(2):
                wait_gather(u, u)
                scale_to(u, rows[u], msgs[u])
                start_scatter(u, u)
                start_gather(u + 2, u)

            def outer(o, _):
                for u in range(2):
                    g = o * 2 + u
                    wait_gather(g, u)
                    wait_scatter(g - 2, u)
                    scale_to(g, rows[u], msgs[u])
                    start_scatter(g, u)
                    start_gather(g + 2, u)
                return 0
            lax.fori_loop(1, PH // 2 - 1, outer, 0)

            for u in range(2):
                g = PH - 2 + u
                wait_gather(g, u)
                wait_scatter(g - 2, u)
                scale_to(g, rows[u], msgs[u])
                start_scatter(g, u)
            for u in range(2):
                wait_scatter(PH - 2 + u, u)

        plsc.subcore_barrier()

        out_base = cid * N_PAD + base_row
        pltpu.sync_copy(acc_sh.at[pl.ds(base_row, ROWS_PER_TILE)],
                        out_hbm.at[pl.ds(out_base, ROWS_PER_TILE)])

    return spmm


_spmm_h = _make_spmm(HH, feature_split=True)
_spmm_c = _make_spmm(CP, feature_split=False)

_BLK = 400
_GRID = N // _BLK


def _mm1_body(x_ref, w_ref, o0_ref, o1_ref):
    d = jnp.dot(x_ref[...], w_ref[...], preferred_element_type=jnp.float32)
    o0_ref[...] = d[:, :HH]
    o1_ref[...] = d[:, HH:]


def _layer2_body(h0_ref, h1_ref, b1_ref, w2_ref, o_ref):
    h = jnp.concatenate([h0_ref[0], h1_ref[0]], axis=1) + b1_ref[...]
    h = jnp.maximum(h, 0.0)
    o_ref[...] = jnp.dot(h, w2_ref[...], preferred_element_type=jnp.float32)


def _out_body(p0_ref, p1_ref, b2_ref, o_ref):
    o = p0_ref[0] + p1_ref[0] + b2_ref[...]
    col = lax.broadcasted_iota(jnp.int32, o.shape, 1)
    o = jnp.where(col < C, o, -jnp.inf)
    m = jnp.max(o, axis=1, keepdims=True)
    ls = jnp.log(jnp.sum(jnp.exp(o - m), axis=1, keepdims=True))
    res = o - m - ls
    o_ref[...] = res[:, :C]


def kernel(x, edge_index, edge_weight, W1, b1, W2, b2):
    dst = edge_index[0]
    src = edge_index[1]
    pad = _E_PAD - E
    src2 = jnp.pad(src, (0, pad)).reshape(-1, K)
    dst2 = jnp.pad(dst, (0, pad)).reshape(-1, K)
    w2 = jnp.pad(edge_weight, (0, pad)).reshape(-1, K)
    wbits = lax.bitcast_convert_type(w2, jnp.int32)
    idx_all = jnp.stack([src2, dst2, wbits], axis=1)
    W2p = jnp.pad(W2, ((0, 0), (0, CP - C)))
    b2p = jnp.pad(b2, (0, CP - C))

    sup0, sup1 = pl.pallas_call(
        _mm1_body,
        grid=(_GRID,),
        in_specs=[
            pl.BlockSpec((_BLK, F_IN), lambda i: (i, 0)),
            pl.BlockSpec((F_IN, H), lambda i: (0, 0)),
        ],
        out_specs=[
            pl.BlockSpec((_BLK, HH), lambda i: (i, 0)),
            pl.BlockSpec((_BLK, HH), lambda i: (i, 0)),
        ],
        out_shape=[
            jax.ShapeDtypeStruct((N, HH), jnp.float32),
            jax.ShapeDtypeStruct((N, HH), jnp.float32),
        ],
    )(x, W1)

    halves1 = _spmm_h(sup0, sup1, idx_all).reshape(NC, N_PAD, HH)

    support2 = pl.pallas_call(
        _layer2_body,
        grid=(_GRID,),
        in_specs=[
            pl.BlockSpec((1, _BLK, HH), lambda i: (0, i, 0)),
            pl.BlockSpec((1, _BLK, HH), lambda i: (1, i, 0)),
            pl.BlockSpec((1, H), lambda i: (0, 0)),
            pl.BlockSpec((H, CP), lambda i: (0, 0)),
        ],
        out_specs=pl.BlockSpec((_BLK, CP), lambda i: (i, 0)),
        out_shape=jax.ShapeDtypeStruct((N, CP), jnp.float32),
    )(halves1, halves1, b1.reshape(1, H), W2p)

    parts2 = _spmm_c(support2, idx_all).reshape(NC, N_PAD, CP)

    out = pl.pallas_call(
        _out_body,
        grid=(_GRID,),
        in_specs=[
            pl.BlockSpec((1, _BLK, CP), lambda i: (0, i, 0)),
            pl.BlockSpec((1, _BLK, CP), lambda i: (1, i, 0)),
            pl.BlockSpec((1, CP), lambda i: (0, 0)),
        ],
        out_specs=pl.BlockSpec((_BLK, C), lambda i: (i, 0)),
        out_shape=jax.ShapeDtypeStruct((N, C), jnp.float32),
    )(parts2, parts2, b2p.reshape(1, CP))

    return out


# speedup vs baseline: 10.6008x; 1.8377x over previous
import functools

import jax
import jax.numpy as jnp
from jax import lax
from jax.experimental import pallas as pl
from jax.experimental.pallas import tpu as pltpu
from jax.experimental.pallas import tpu_sc as plsc

N = 10000
E = 320000
F_IN = 128
H = 128
HH = H // 2
C = 40
CP = 48

NC = 2
NS = 16
NW = NC * NS
K = 128
PH = 80

N_PAD = 10112
ROWS_PER_TILE = N_PAD // NS

_E_PAD = NS * 2 * PH * K


def _make_spmm(feat: int, feature_split: bool, ph: int, phases: int):
    mesh = plsc.VectorSubcoreMesh(core_axis_name="c", subcore_axis_name="s")
    n_sup = 2 if feature_split else 1

    @functools.partial(
        pl.kernel,
        out_type=jax.ShapeDtypeStruct((NC * N_PAD, feat), jnp.float32),
        mesh=mesh,
        scratch_types=[
            pltpu.VMEM((K, feat), jnp.float32),
            pltpu.VMEM((K, feat), jnp.float32),
            pltpu.VMEM((K, feat), jnp.float32),
            pltpu.VMEM((K, feat), jnp.float32),
            pltpu.VMEM((ph, 3, K), jnp.int32),
            pltpu.VMEM_SHARED((N_PAD, feat), jnp.float32),
            pltpu.VMEM_SHARED((N_PAD, feat), jnp.float32),
            pltpu.SemaphoreType.DMA,
            pltpu.SemaphoreType.DMA,
            pltpu.SemaphoreType.DMA,
            pltpu.SemaphoreType.DMA,
            pltpu.SemaphoreType.DMA,
        ],
        compiler_params=pltpu.CompilerParams(use_tc_tiling_on_sc=False,
                                             needs_layout_passes=False),
    )
    def spmm(*args):
        sups = args[:n_sup]
        idx_hbm = args[n_sup]
        out_hbm = args[n_sup + 1]
        (rows0, rows1, msgs0, msgs1, idxb, acc_sh, sup_sh,
         gs0, gs1, ss0, ss1, isem) = args[n_sup + 2:]
        rows = (rows0, rows1)
        msgs = (msgs0, msgs1)
        gsem = (gs0, gs1)
        ssem = (ss0, ss1)
        cid = lax.axis_index("c")
        sid = lax.axis_index("s")

        if feature_split:
            cb = sid * (phases * ph)
        else:
            cb = (sid * NC + cid) * ph

        cp_i = pltpu.async_copy(idx_hbm.at[pl.ds(cb, ph)], idxb, isem)

        base_row = sid * ROWS_PER_TILE
        stripe = pl.ds(base_row, ROWS_PER_TILE)

        if feature_split:
            @pl.when(cid == 0)
            def _():
                pltpu.async_copy(sups[0].at[stripe], sup_sh.at[stripe], gs0)

            @pl.when(cid == 1)
            def _():
                pltpu.async_copy(sups[1].at[stripe], sup_sh.at[stripe], gs0)
        else:
            pltpu.async_copy(sups[0].at[stripe], sup_sh.at[stripe], gs0)

        def zero_row(i, _):
            for f in range(feat // 16):
                msgs0[i, pl.ds(f * 16, 16)] = jnp.zeros((16,), jnp.float32)
            return 0
        lax.fori_loop(0, K, zero_row, 0)

        nfull = ROWS_PER_TILE // K
        rem = ROWS_PER_TILE - nfull * K
        for j in range(nfull):
            pltpu.sync_copy(msgs0, acc_sh.at[pl.ds(base_row + j * K, K)])
        if rem:
            pltpu.sync_copy(msgs0.at[pl.ds(0, rem)],
                            acc_sh.at[pl.ds(base_row + nfull * K, rem)])
        cp_i.wait()
        pltpu.make_async_copy(sups[0].at[stripe], sup_sh.at[stripe],
                              gs0).wait()
        plsc.subcore_barrier()

        def scale_to(g, rv, mv):
            def scale16(b16, _):
                w16 = plsc.bitcast(idxb[g, 2, pl.ds(b16 * 16, 16)],
                                   jnp.float32)
                e0 = b16 * 16
                for i in range(16):
                    wv = jnp.full((16,), w16[i], jnp.float32)
                    for f in range(feat // 16):
                        sl = pl.ds(f * 16, 16)
                        mv[e0 + i, sl] = rv[e0 + i, sl] * wv
                return 0
            lax.fori_loop(0, K // 16, scale16, 0)

        def start_gather(g, b):
            pltpu.async_copy(sup_sh.at[idxb.at[g, 0]], rows[b], gsem[b])

        def wait_gather(g, b):
            pltpu.make_async_copy(sup_sh.at[idxb.at[g, 0]], rows[b],
                                  gsem[b]).wait()

        def start_scatter(g, m):
            pltpu.async_copy(msgs[m], acc_sh.at[idxb.at[g, 1]], ssem[m],
                             add=True)

        def wait_scatter(g, m):
            pltpu.make_async_copy(msgs[m], acc_sh.at[idxb.at[g, 1]],
                                  ssem[m]).wait()

        for p in range(phases):
            if p > 0:
                pltpu.sync_copy(idx_hbm.at[pl.ds(cb + p * ph, ph)], idxb)

            for b in range(2):
                start_gather(b, b)

            for u in range(2):
                wait_gather(u, u)
                scale_to(u, rows[u], msgs[u])
                start_scatter(u, u)
                start_gather(u + 2, u)

            def outer(o, _):
                for u in range(2):
                    g = o * 2 + u
                    wait_gather(g, u)
                    wait_scatter(g - 2, u)
                    scale_to(g, rows[u], msgs[u])
                    start_scatter(g, u)
                    start_gather(g + 2, u)
                return 0
            lax.fori_loop(1, ph // 2 - 1, outer, 0)

            for u in range(2):
                g = ph - 2 + u
                wait_gather(g, u)
                wait_scatter(g - 2, u)
                scale_to(g, rows[u], msgs[u])
                start_scatter(g, u)
            for u in range(2):
                wait_scatter(ph - 2 + u, u)

        plsc.subcore_barrier()

        out_base = cid * N_PAD + base_row
        pltpu.sync_copy(acc_sh.at[stripe],
                        out_hbm.at[pl.ds(out_base, ROWS_PER_TILE)])

    return spmm


_spmm_h = _make_spmm(HH, feature_split=True, ph=40, phases=4)
_spmm_c = _make_spmm(CP, feature_split=False, ph=80, phases=1)

_BLK = 400
_GRID = N // _BLK


def _mm1_body(x_ref, w_ref, o0_ref, o1_ref):
    d = jnp.dot(x_ref[...], w_ref[...], preferred_element_type=jnp.float32)
    o0_ref[...] = d[:, :HH]
    o1_ref[...] = d[:, HH:]


def _layer2_body(h0_ref, h1_ref, b1_ref, w2_ref, o_ref):
    h = jnp.concatenate([h0_ref[0], h1_ref[0]], axis=1) + b1_ref[...]
    h = jnp.maximum(h, 0.0)
    o_ref[...] = jnp.dot(h, w2_ref[...], preferred_element_type=jnp.float32)


def _out_body(p0_ref, p1_ref, b2_ref, o_ref):
    o = p0_ref[0] + p1_ref[0] + b2_ref[...]
    col = lax.broadcasted_iota(jnp.int32, o.shape, 1)
    o = jnp.where(col < C, o, -jnp.inf)
    m = jnp.max(o, axis=1, keepdims=True)
    ls = jnp.log(jnp.sum(jnp.exp(o - m), axis=1, keepdims=True))
    res = o - m - ls
    o_ref[...] = res[:, :C]


def kernel(x, edge_index, edge_weight, W1, b1, W2, b2):
    dst = edge_index[0]
    src = edge_index[1]
    pad = _E_PAD - E
    src2 = jnp.pad(src, (0, pad)).reshape(-1, K)
    dst2 = jnp.pad(dst, (0, pad)).reshape(-1, K)
    w2 = jnp.pad(edge_weight, (0, pad)).reshape(-1, K)
    wbits = lax.bitcast_convert_type(w2, jnp.int32)
    idx_all = jnp.stack([src2, dst2, wbits], axis=1)
    W2p = jnp.pad(W2, ((0, 0), (0, CP - C)))
    b2p = jnp.pad(b2, (0, CP - C))

    sup0, sup1 = pl.pallas_call(
        _mm1_body,
        grid=(_GRID,),
        in_specs=[
            pl.BlockSpec((_BLK, F_IN), lambda i: (i, 0)),
            pl.BlockSpec((F_IN, H), lambda i: (0, 0)),
        ],
        out_specs=[
            pl.BlockSpec((_BLK, HH), lambda i: (i, 0)),
            pl.BlockSpec((_BLK, HH), lambda i: (i, 0)),
        ],
        out_shape=[
            jax.ShapeDtypeStruct((N, HH), jnp.float32),
            jax.ShapeDtypeStruct((N, HH), jnp.float32),
        ],
    )(x, W1)

    sup0p = jnp.pad(sup0, ((0, N_PAD - N), (0, 0)))
    sup1p = jnp.pad(sup1, ((0, N_PAD - N), (0, 0)))
    halves1 = _spmm_h(sup0p, sup1p, idx_all).reshape(NC, N_PAD, HH)

    support2 = pl.pallas_call(
        _layer2_body,
        grid=(_GRID,),
        in_specs=[
            pl.BlockSpec((1, _BLK, HH), lambda i: (0, i, 0)),
            pl.BlockSpec((1, _BLK, HH), lambda i: (1, i, 0)),
            pl.BlockSpec((1, H), lambda i: (0, 0)),
            pl.BlockSpec((H, CP), lambda i: (0, 0)),
        ],
        out_specs=pl.BlockSpec((_BLK, CP), lambda i: (i, 0)),
        out_shape=jax.ShapeDtypeStruct((N, CP), jnp.float32),
    )(halves1, halves1, b1.reshape(1, H), W2p)

    support2p = jnp.pad(support2, ((0, N_PAD - N), (0, 0)))
    parts2 = _spmm_c(support2p, idx_all).reshape(NC, N_PAD, CP)

    out = pl.pallas_call(
        _out_body,
        grid=(_GRID,),
        in_specs=[
            pl.BlockSpec((1, _BLK, CP), lambda i: (0, i, 0)),
            pl.BlockSpec((1, _BLK, CP), lambda i: (1, i, 0)),
            pl.BlockSpec((1, CP), lambda i: (0, 0)),
        ],
        out_specs=pl.BlockSpec((_BLK, C), lambda i: (i, 0)),
        out_shape=jax.ShapeDtypeStruct((N, C), jnp.float32),
    )(parts2, parts2, b2p.reshape(1, CP))

    return out


# speedup vs baseline: 11.4615x; 1.0812x over previous
import functools

import jax
import jax.numpy as jnp
from jax import lax
from jax.experimental import pallas as pl
from jax.experimental.pallas import tpu as pltpu
from jax.experimental.pallas import tpu_sc as plsc

N = 10000
E = 320000
F_IN = 128
H = 128
HH = F_IN // 2
C = 40
CP = 48

NC = 2
NS = 16
K = 128

N_PAD = 10112
ROWS_PER_TILE = N_PAD // NS
LAST_ROWS = N - (NS - 1) * ROWS_PER_TILE

_E_PAD = NS * 2 * 80 * K


def _make_spmm(feat: int, feature_split: bool, ph: int, phases: int):
    mesh = plsc.VectorSubcoreMesh(core_axis_name="c", subcore_axis_name="s")

    @functools.partial(
        pl.kernel,
        out_type=jax.ShapeDtypeStruct((NC, N_PAD, feat), jnp.float32),
        mesh=mesh,
        scratch_types=[
            pltpu.VMEM((K, feat), jnp.float32),
            pltpu.VMEM((K, feat), jnp.float32),
            pltpu.VMEM((K, feat), jnp.float32),
            pltpu.VMEM((K, feat), jnp.float32),
            pltpu.VMEM((ph, 3, K), jnp.int32),
            pltpu.VMEM_SHARED((N_PAD, feat), jnp.float32),
            pltpu.VMEM_SHARED((N_PAD, feat), jnp.float32),
            pltpu.SemaphoreType.DMA,
            pltpu.SemaphoreType.DMA,
            pltpu.SemaphoreType.DMA,
            pltpu.SemaphoreType.DMA,
            pltpu.SemaphoreType.DMA,
        ],
        compiler_params=pltpu.CompilerParams(use_tc_tiling_on_sc=False,
                                             needs_layout_passes=False),
    )
    def spmm(sup_hbm, idx_hbm, out_hbm,
             rows0, rows1, msgs0, msgs1, idxb, acc_sh, sup_sh,
             gs0, gs1, ss0, ss1, isem):
        rows = (rows0, rows1)
        msgs = (msgs0, msgs1)
        gsem = (gs0, gs1)
        ssem = (ss0, ss1)
        cid = lax.axis_index("c")
        sid = lax.axis_index("s")

        if feature_split:
            cb = sid * (phases * ph)
        else:
            cb = (sid * NC + cid) * ph

        cp_i = pltpu.async_copy(idx_hbm.at[pl.ds(cb, ph)], idxb, isem)

        base_row = sid * ROWS_PER_TILE
        stripe = pl.ds(base_row, ROWS_PER_TILE)

        def stage_rows(nrows):
            rs = pl.ds(base_row, nrows)
            if feature_split:
                @pl.when(cid == 0)
                def _():
                    pltpu.async_copy(sup_hbm.at[rs, pl.ds(0, feat)],
                                     sup_sh.at[rs], gs0)

                @pl.when(cid == 1)
                def _():
                    pltpu.async_copy(sup_hbm.at[rs, pl.ds(feat, feat)],
                                     sup_sh.at[rs], gs0)
            else:
                pltpu.async_copy(sup_hbm.at[rs], sup_sh.at[rs], gs0)

        @pl.when(sid < NS - 1)
        def _():
            stage_rows(ROWS_PER_TILE)

        @pl.when(sid == NS - 1)
        def _():
            stage_rows(LAST_ROWS)

        def zero_row(i, _):
            for f in range(feat // 16):
                msgs0[i, pl.ds(f * 16, 16)] = jnp.zeros((16,), jnp.float32)
            return 0
        lax.fori_loop(0, K, zero_row, 0)

        nfull = ROWS_PER_TILE // K
        rem = ROWS_PER_TILE - nfull * K
        for j in range(nfull):
            pltpu.sync_copy(msgs0, acc_sh.at[pl.ds(base_row + j * K, K)])
        if rem:
            pltpu.sync_copy(msgs0.at[pl.ds(0, rem)],
                            acc_sh.at[pl.ds(base_row + nfull * K, rem)])
        cp_i.wait()
        @pl.when(sid < NS - 1)
        def _():
            pltpu.make_async_copy(
                sup_sh.at[stripe], sup_sh.at[stripe], gs0).wait()

        @pl.when(sid == NS - 1)
        def _():
            pltpu.make_async_copy(
                sup_sh.at[pl.ds(base_row, LAST_ROWS)],
                sup_sh.at[pl.ds(base_row, LAST_ROWS)], gs0).wait()

        plsc.subcore_barrier()

        def scale_to(g, rv, mv):
            def scale16(b16, _):
                w16 = plsc.bitcast(idxb[g, 2, pl.ds(b16 * 16, 16)],
                                   jnp.float32)
                e0 = b16 * 16
                for i in range(16):
                    wv = jnp.full((16,), w16[i], jnp.float32)
                    for f in range(feat // 16):
                        sl = pl.ds(f * 16, 16)
                        mv[e0 + i, sl] = rv[e0 + i, sl] * wv
                return 0
            lax.fori_loop(0, K // 16, scale16, 0)

        def start_gather(g, b):
            pltpu.async_copy(sup_sh.at[idxb.at[g, 0]], rows[b], gsem[b])

        def wait_gather(g, b):
            pltpu.make_async_copy(sup_sh.at[idxb.at[g, 0]], rows[b],
                                  gsem[b]).wait()

        def start_scatter(g, m):
            pltpu.async_copy(msgs[m], acc_sh.at[idxb.at[g, 1]], ssem[m],
                             add=True)

        def wait_scatter(g, m):
            pltpu.make_async_copy(msgs[m], acc_sh.at[idxb.at[g, 1]],
                                  ssem[m]).wait()

        for p in range(phases):
            if p > 0:
                pltpu.sync_copy(idx_hbm.at[pl.ds(cb + p * ph, ph)], idxb)

            for b in range(2):
                start_gather(b, b)

            for u in range(2):
                wait_gather(u, u)
                scale_to(u, rows[u], msgs[u])
                start_scatter(u, u)
                start_gather(u + 2, u)

            def outer(o, _):
                for u in range(2):
                    g = o * 2 + u
                    wait_gather(g, u)
                    wait_scatter(g - 2, u)
                    scale_to(g, rows[u], msgs[u])
                    start_scatter(g, u)
                    start_gather(g + 2, u)
                return 0
            lax.fori_loop(1, ph // 2 - 1, outer, 0)

            for u in range(2):
                g = ph - 2 + u
                wait_gather(g, u)
                wait_scatter(g - 2, u)
                scale_to(g, rows[u], msgs[u])
                start_scatter(g, u)
            for u in range(2):
                wait_scatter(ph - 2 + u, u)

        plsc.subcore_barrier()

        pltpu.sync_copy(acc_sh.at[stripe], out_hbm.at[cid, stripe])

    return spmm


_spmm_h = _make_spmm(HH, feature_split=True, ph=40, phases=4)
_spmm_c = _make_spmm(CP, feature_split=False, ph=80, phases=1)

_BLK = 400
_GRID = N // _BLK


def _layer2_body(a0_ref, a1_ref, w1_ref, b1_ref, w2_ref, o_ref):
    agg = jnp.concatenate([a0_ref[0], a1_ref[0]], axis=1)
    h = jnp.dot(agg, w1_ref[...], preferred_element_type=jnp.float32)
    h = jnp.maximum(h + b1_ref[...], 0.0)
    o_ref[...] = jnp.dot(h, w2_ref[...], preferred_element_type=jnp.float32)


def _out_body(p0_ref, p1_ref, b2_ref, o_ref):
    o = p0_ref[0] + p1_ref[0] + b2_ref[...]
    col = lax.broadcasted_iota(jnp.int32, o.shape, 1)
    o = jnp.where(col < C, o, -jnp.inf)
    m = jnp.max(o, axis=1, keepdims=True)
    ls = jnp.log(jnp.sum(jnp.exp(o - m), axis=1, keepdims=True))
    res = o - m - ls
    o_ref[...] = res[:, :C]


def kernel(x, edge_index, edge_weight, W1, b1, W2, b2):
    dst = edge_index[0]
    src = edge_index[1]
    pad = _E_PAD - E
    src2 = jnp.pad(src, (0, pad)).reshape(-1, K)
    dst2 = jnp.pad(dst, (0, pad)).reshape(-1, K)
    w2 = jnp.pad(edge_weight, (0, pad)).reshape(-1, K)
    wbits = lax.bitcast_convert_type(w2, jnp.int32)
    idx_all = jnp.stack([src2, dst2, wbits], axis=1)
    W2p = jnp.pad(W2, ((0, 0), (0, CP - C)))
    b2p = jnp.pad(b2, (0, CP - C))

    agg1 = _spmm_h(x, idx_all)

    support2 = pl.pallas_call(
        _layer2_body,
        grid=(_GRID,),
        in_specs=[
            pl.BlockSpec((1, _BLK, HH), lambda i: (0, i, 0)),
            pl.BlockSpec((1, _BLK, HH), lambda i: (1, i, 0)),
            pl.BlockSpec((F_IN, H), lambda i: (0, 0)),
            pl.BlockSpec((1, H), lambda i: (0, 0)),
            pl.BlockSpec((H, CP), lambda i: (0, 0)),
        ],
        out_specs=pl.BlockSpec((_BLK, CP), lambda i: (i, 0)),
        out_shape=jax.ShapeDtypeStruct((N, CP), jnp.float32),
    )(agg1, agg1, W1, b1.reshape(1, H), W2p)

    parts2 = _spmm_c(support2, idx_all)

    out = pl.pallas_call(
        _out_body,
        grid=(_GRID,),
        in_specs=[
            pl.BlockSpec((1, _BLK, CP), lambda i: (0, i, 0)),
            pl.BlockSpec((1, _BLK, CP), lambda i: (1, i, 0)),
            pl.BlockSpec((1, CP), lambda i: (0, 0)),
        ],
        out_specs=pl.BlockSpec((_BLK, C), lambda i: (i, 0)),
        out_shape=jax.ShapeDtypeStruct((N, C), jnp.float32),
    )(parts2, parts2, b2p.reshape(1, CP))

    return out
